# baseline (speedup 1.0000x reference)
import functools

import jax
import jax.numpy as jnp
from jax import lax
from jax.experimental import pallas as pl
from jax.experimental.pallas import tpu as pltpu
from jax.experimental.pallas import tpu_sc as plsc

_N = 25000
_NP = 25088
_RPT = _NP // 16
_D = 128
_HD = 64
_E = 400000
_CH = 128
_EP = 401408
_EPT = _EP // 16
_NCH = _EPT // _CH
_NB = 1024
_BAG = 16
_LLM_D = 32000
_KB = 3200
_NKB = _LLM_D // _KB

def _get_mesh():
    return plsc.VectorSubcoreMesh(core_axis_name="c", subcore_axis_name="s",
                                  num_cores=2, num_subcores=16)


@functools.lru_cache(maxsize=None)
def _make_segsum(group_phases):
    n_in = sum(2 + 4 * p for p in group_phases)
    n_out = 2 * len(group_phases)
    nbuf = 2
    sec = 14
    nsec = _NCH // sec

    @functools.partial(
        pl.kernel,
        out_type=[jax.ShapeDtypeStruct((_NP, _HD), jnp.float32)] * n_out,
        mesh=_get_mesh(),
        compiler_params=pltpu.CompilerParams(use_tc_tiling_on_sc=False),
        scratch_types=[
            pltpu.VMEM((sec, _CH), jnp.int32),
            pltpu.VMEM((sec, _CH), jnp.int32),
        ]
        + [pltpu.VMEM((_CH, _HD), jnp.float32) for _ in range(nbuf)]
        + [pltpu.SemaphoreType.DMA for _ in range(nbuf)]
        + [pltpu.VMEM_SHARED((_NP, _HD), jnp.float32)],
    )
    def segsum(*refs):
        ins = refs[:n_in]
        outs = refs[n_in:n_in + n_out]
        src_v, dst_v = refs[n_in + n_out], refs[n_in + n_out + 1]
        bufs = refs[n_in + n_out + 2:n_in + n_out + 2 + nbuf]
        sems = refs[n_in + n_out + 2 + nbuf:n_in + n_out + 2 + 2 * nbuf]
        acc = refs[n_in + n_out + 2 + 2 * nbuf]
        c = lax.axis_index("c")
        t = lax.axis_index("s")
        r0 = t * _RPT

        def run(init, phase_refs, out):
            pltpu.sync_copy(init.at[pl.ds(r0, _RPT)], acc.at[pl.ds(r0, _RPT)])
            plsc.subcore_barrier()
            for tab, src, dst in phase_refs:

                def section(s, _):
                    pltpu.sync_copy(src.at[t, pl.ds(s * sec, sec)], src_v)
                    pltpu.sync_copy(dst.at[t, pl.ds(s * sec, sec)], dst_v)

                    def pair(k, _):
                        cp0 = pltpu.async_copy(
                            tab.at[src_v.at[2 * k]], bufs[0], sems[0])
                        cp1 = pltpu.async_copy(
                            tab.at[src_v.at[2 * k + 1]], bufs[1], sems[1])
                        cp0.wait()
                        pltpu.sync_copy(bufs[0], acc.at[dst_v.at[2 * k]],
                                        add=True)
                        cp1.wait()
                        pltpu.sync_copy(bufs[1], acc.at[dst_v.at[2 * k + 1]],
                                        add=True)
                        return 0

                    lax.fori_loop(0, sec // 2, pair, 0)
                    return 0

                lax.fori_loop(0, nsec, section, 0)
            plsc.subcore_barrier()
            pltpu.sync_copy(acc.at[pl.ds(r0, _RPT)], out.at[pl.ds(r0, _RPT)])

        def side(half):
            base = 0
            for g, np_ in enumerate(group_phases):
                init = ins[base + half]
                phase_refs = [
                    (ins[base + 2 + 4 * p + half],
                     ins[base + 2 + 4 * p + 2],
                     ins[base + 2 + 4 * p + 3])
                    for p in range(np_)]
                run(init, phase_refs, outs[2 * g + half])
                base += 2 + 4 * np_

        @pl.when(c == 0)
        def _():
            side(0)

        @pl.when(c == 1)
        def _():
            side(1)

    return segsum


_MPW = (_NB * _BAG) // 32
_MPCH = _MPW // _CH


@functools.lru_cache(maxsize=None)
def _make_mp_gather():
    @functools.partial(
        pl.kernel,
        out_type=[jax.ShapeDtypeStruct((_NB * _BAG, _D), jnp.float32),
                  jax.ShapeDtypeStruct((_NB * _BAG, _D), jnp.float32)],
        mesh=_get_mesh(),
        scratch_types=[
            pltpu.VMEM((_CH,), jnp.int32),
            pltpu.VMEM((_CH, _D), jnp.float32),
            pltpu.SemaphoreType.DMA,
        ],
    )
    def mp_gather(hdf, hsf, idxd, idxs, gd, gs, idx_v, rows_v, sem):
        c = lax.axis_index("c")
        s = lax.axis_index("s")
        wid = s * 2 + c

        def one(tab, idx, out):
            def chunk(i, _):
                off = wid * _MPW + i * _CH
                pltpu.sync_copy(idx.at[pl.ds(off, _CH)], idx_v)
                pltpu.async_copy(tab.at[idx_v], rows_v, sem).wait()
                pltpu.sync_copy(rows_v, out.at[pl.ds(off, _CH)])
                return 0

            lax.fori_loop(0, _MPCH, chunk, 0)

        one(hdf, idxd, gd)
        one(hsf, idxs, gs)

    return mp_gather


def _dot(a, b):
    return jnp.dot(a, b, preferred_element_type=jnp.float32)


def _row_spec(r, ncols=_D):
    return pl.BlockSpec((r, ncols), lambda i: (i, 0))


def _full_spec(shape):
    return pl.BlockSpec(shape, lambda i: tuple(0 for _ in shape))


def _mm_body_first(xd, xs, wld, bld, wls, bls, wdd, wrd, wrr, wsd, wss,
                   h0d, h0s, *outs):
    d = jnp.maximum(_dot(xd[...], wld[...]) + bld[...], 0.0)
    s = jnp.maximum(_dot(xs[...], wls[...]) + bls[...], 0.0)
    h0d[...] = d
    h0s[...] = s
    _mm_common(d, s, wdd, wrd, wrr, wsd, wss, outs)


def _layer_mm_first(xd, xs, wld, bld, wls, bls, wdd, wrd, wrr, wsd, wss):
    return pl.pallas_call(
        _mm_body_first,
        grid=(16,),
        in_specs=[_row_spec(_RPT)] * 2
        + [_full_spec((_D, _D)), _full_spec((1, _D)),
           _full_spec((_D, _D)), _full_spec((1, _D))]
        + [_full_spec((_D, _D))] * 5,
        out_specs=[_row_spec(_RPT)] * 2 + [_row_spec(_RPT, _HD)] * 10,
        out_shape=[jax.ShapeDtypeStruct((_NP, _D), jnp.float32)] * 2
        + [jax.ShapeDtypeStruct((_NP, _HD), jnp.float32)] * 10,
    )(xd, xs, wld, bld, wls, bls, wdd, wrd, wrr, wsd, wss)


def _halves_in(hd, hs):
    def load(x):
        if isinstance(x, tuple):
            return jnp.concatenate(
                [jnp.maximum(x[0][...], 0.0), jnp.maximum(x[1][...], 0.0)],
                axis=1)
        return x[...]
    return load(hd), load(hs)


def _mm_body_full(hd, hs, wdd, wrd, wrr, wsd, wss, *outs):
    _mm_common(hd[...], hs[...], wdd, wrd, wrr, wsd, wss, outs)


def _mm_body_halves(hdl, hdr, hsl, hsr, wdd, wrd, wrr, wsd, wss, *outs):
    d, s = _halves_in((hdl, hdr), (hsl, hsr))
    _mm_common(d, s, wdd, wrd, wrr, wsd, wss, outs)


def _mm_common(d, s, wdd, wrd, wrr, wsd, wss, outs):
    vals = [_dot(s, wdd[...]), _dot(d, wrd[...]), _dot(d, wrr[...]),
            _dot(d, wsd[...]), _dot(s, wss[...])]
    for j, v in enumerate(vals):
        outs[2 * j][...] = v[:, :_HD]
        outs[2 * j + 1][...] = v[:, _HD:]


def _layer_mm(hd, hs, wdd, wrd, wrr, wsd, wss):
    halved = isinstance(hd, tuple)
    body = _mm_body_halves if halved else _mm_body_full
    h_in = ([_row_spec(_RPT, _HD)] * 4 if halved
            else [_row_spec(_RPT), _row_spec(_RPT)])
    h_args = (hd + hs) if halved else (hd, hs)
    return pl.pallas_call(
        body,
        grid=(16,),
        in_specs=h_in + [_full_spec((_D, _D))] * 5,
        out_specs=[_row_spec(_RPT, _HD)] * 10,
        out_shape=[jax.ShapeDtypeStruct((_NP, _HD), jnp.float32)] * 10,
    )(*h_args, wdd, wrd, wrr, wsd, wss)


def _attn_pool(hs, wa, va):
    va_col = va[...].reshape(_D, 1)
    es = [_dot(jnp.tanh(_dot(h, wa[...])), va_col) for h in hs]
    m = jnp.maximum(jnp.maximum(es[0], es[1]), es[2])
    ws = [jnp.exp(e - m) for e in es]
    tot = ws[0] + ws[1] + ws[2]
    return (ws[0] * hs[0] + ws[1] * hs[1] + ws[2] * hs[2]) / tot


def _attn_body(d0, d1l, d1r, d2l, d2r, s0, s1l, s1r, s2l, s2r,
               wad, vad, was, vas, od, os_):
    d1, s1 = _halves_in((d1l, d1r), (s1l, s1r))
    d2, s2 = _halves_in((d2l, d2r), (s2l, s2r))
    od[...] = _attn_pool([d0[...], d1, d2], wad, vad)
    os_[...] = _attn_pool([s0[...], s1, s2], was, vas)


def _layer_attn(d0, d1, d2, s0, s1, s2, wad, vad, was, vas):
    return pl.pallas_call(
        _attn_body,
        grid=(16,),
        in_specs=[_row_spec(_RPT)] + [_row_spec(_RPT, _HD)] * 4
        + [_row_spec(_RPT)] + [_row_spec(_RPT, _HD)] * 4
        + [_full_spec((_D, _D)), _full_spec((1, _D)),
           _full_spec((_D, _D)), _full_spec((1, _D))],
        out_specs=[_row_spec(_RPT)] * 2,
        out_shape=[jax.ShapeDtypeStruct((_NP, _D), jnp.float32)] * 2,
    )(d0, *d1, *d2, s0, *s1, *s2, wad, vad, was, vas)


_MB = 256


def _mil_body(gd, gs, wagg, vmil, wmil, attn_o, bag_o):
    g = (gd[...] + gs[...]).reshape(_MB * _BAG, _D)
    ins = jnp.maximum(_dot(g, wagg[...]), 0.0)
    t3 = jnp.tanh(_dot(ins, vmil[...])).reshape(_MB, _BAG, _D)
    ins3 = ins.reshape(_MB, _BAG, _D)
    w_col = wmil[...].reshape(_D, 1)
    cols = [_dot(t3[:, k, :], w_col) for k in range(_BAG)]
    al = jnp.concatenate(cols, axis=1)
    m = jnp.max(al, axis=1, keepdims=True)
    e = jnp.exp(al - m)
    attn = e / jnp.sum(e, axis=1, keepdims=True)
    attn_o[...] = attn
    bag = attn[:, 0:1] * ins3[:, 0, :]
    for k in range(1, _BAG):
        bag = bag + attn[:, k:k + 1] * ins3[:, k, :]
    bag_o[...] = bag


def _mil(gd3, gs3, wagg, vmil, wmil):
    return pl.pallas_call(
        _mil_body,
        grid=(_NB // _MB,),
        in_specs=[pl.BlockSpec((_MB, _BAG, _D), lambda i: (i, 0, 0))] * 2
        + [_full_spec((_D, _D)), _full_spec((_D, _D)), _full_spec((1, _D))],
        out_specs=[pl.BlockSpec((_MB, _BAG), lambda i: (i, 0)),
                   pl.BlockSpec((_MB, _D), lambda i: (i, 0))],
        out_shape=[jax.ShapeDtypeStruct((_NB, _BAG), jnp.float32),
                   jax.ShapeDtypeStruct((_NB, _D), jnp.float32)],
    )(gd3, gs3, wagg, vmil, wmil)


def _llm_body(x, w, b, bag, w1, b1, w2, b2, o, pred):
    k = pl.program_id(0)

    @pl.when(k == 0)
    def _():
        o[...] = jnp.broadcast_to(b[...], (_NB, _D))

    o[...] += _dot(x[...], w[...])

    @pl.when(k == _NKB - 1)
    def _():
        y = o[...]
        n = jnp.sqrt(jnp.sum(y * y, axis=1, keepdims=True)) + 1e-12
        llm_n = y / n
        kg = bag[...]
        nk = jnp.sqrt(jnp.sum(kg * kg, axis=1, keepdims=True)) + 1e-12
        kgn = kg / nk
        w1v = w1[...]
        h = _dot(kgn, w1v[:_D, :]) + _dot(llm_n, w1v[_D:, :]) + b1[...]
        h = jnp.maximum(h, 0.0)
        p = _dot(h, w2[...]) + b2[0, 0]
        pred[...] = jnp.broadcast_to(p, (_NB, _D))


def _llm_head(x, w, b, bag, w1, b1, w2, b2):
    return pl.pallas_call(
        _llm_body,
        grid=(_NKB,),
        in_specs=[pl.BlockSpec((_NB, _KB), lambda k: (0, k)),
                  pl.BlockSpec((_KB, _D), lambda k: (k, 0)),
                  _full_spec((1, _D)),
                  pl.BlockSpec((_NB, _D), lambda k: (0, 0)),
                  pl.BlockSpec((2 * _D, _D), lambda k: (0, 0)),
                  _full_spec((1, _D)),
                  pl.BlockSpec((_D, 1), lambda k: (0, 0)),
                  _full_spec((1, 1))],
        out_specs=[pl.BlockSpec((_NB, _D), lambda k: (0, 0)),
                   pl.BlockSpec((_NB, _D), lambda k: (0, 0))],
        out_shape=[jax.ShapeDtypeStruct((_NB, _D), jnp.float32),
                   jax.ShapeDtypeStruct((_NB, _D), jnp.float32)],
    )(x, w, b, bag, w1, b1, w2, b2)


def kernel(drug_feat, disease_feat, edge_dd, edge_rd, edge_rr, mp_ins,
           llm_rep, W_lin_drug, b_lin_drug, W_lin_dis, b_lin_dis, W_dd,
           W_rd, W_rr, W_self_drug, W_self_dis, Wa_drug, va_drug, Wa_dis,
           va_dis, W_agg, V_mil, w_mil, W_llm, b_llm, W_mlp1, b_mlp1,
           W_mlp2, b_mlp2):
    rowpad = ((0, _NP - _N), (0, 0))
    dfp = jnp.pad(drug_feat, rowpad)
    sfp = jnp.pad(disease_feat, rowpad)

    def pad_edges(e):
        src = jnp.pad(e[0], (0, _EP - _E)).reshape(16, _NCH, _CH)
        dst = jnp.pad(e[1], (0, _EP - _E),
                      constant_values=_N + 80).reshape(16, _NCH, _CH)
        return src, dst

    dd_s, dd_d = pad_edges(edge_dd)
    rd_s, rd_d = pad_edges(edge_rd)
    rr_s, rr_d = pad_edges(edge_rr)

    row = lambda v: v.reshape(1, -1)

    drugs = []
    diss = []
    hd = hs = None
    for l in range(2):
        if l == 0:
            (h0d, h0s, tddL, tddR, trdL, trdR, trrL, trrR,
             sdL, sdR, ssL, ssR) = _layer_mm_first(
                dfp, sfp, W_lin_drug, row(b_lin_drug),
                W_lin_dis, row(b_lin_dis),
                W_dd[0], W_rd[0], W_rr[0], W_self_drug[0], W_self_dis[0])
            drugs.append(h0d)
            diss.append(h0s)
        else:
            (tddL, tddR, trdL, trdR, trrL, trrR,
             sdL, sdR, ssL, ssR) = _layer_mm(
                hd, hs, W_dd[l], W_rd[l], W_rr[l],
                W_self_drug[l], W_self_dis[l])
        msL, msR, mdL, mdR = _make_segsum((2, 1))(
            ssL, ssR, tddL, tddR, dd_s, dd_d, trdL, trdR, rd_s, rd_d,
            sdL, sdR, trrL, trrR, rr_s, rr_d)
        hd, hs = (mdL, mdR), (msL, msR)
        drugs.append(hd)
        diss.append(hs)

    hdf, hsf = _layer_attn(drugs[0], drugs[1], drugs[2],
                           diss[0], diss[1], diss[2],
                           Wa_drug, row(va_drug), Wa_dis, row(va_dis))

    idxd = mp_ins[..., 0].reshape(-1)
    idxs = mp_ins[..., 1].reshape(-1)
    gd, gs = _make_mp_gather()(hdf, hsf, idxd, idxs)

    attn, bag = _mil(gd.reshape(_NB, _BAG, _D), gs.reshape(_NB, _BAG, _D),
                     W_agg, V_mil, row(w_mil))

    _, pred_full = _llm_head(llm_rep, W_llm, row(b_llm), bag,
                             W_mlp1, row(b_mlp1), W_mlp2,
                             b_mlp2.reshape(1, 1))
    return pred_full[:, :1], attn


# speedup vs baseline: 1.0586x; 1.0586x over previous
import functools

import jax
import jax.numpy as jnp
from jax import lax
from jax.experimental import pallas as pl
from jax.experimental.pallas import tpu as pltpu
from jax.experimental.pallas import tpu_sc as plsc

_N = 25000
_NP = 25088
_RPT = _NP // 16
_D = 128
_HD = 64
_E = 400000
_CH = 128
_EP = 401408
_EPT = _EP // 16
_NCH = _EPT // _CH
_NB = 1024
_BAG = 16
_LLM_D = 32000
_KB = 3200
_NKB = _LLM_D // _KB

def _get_mesh():
    return plsc.VectorSubcoreMesh(core_axis_name="c", subcore_axis_name="s",
                                  num_cores=2, num_subcores=16)


@functools.lru_cache(maxsize=None)
def _make_segsum(group_phases):
    n_in = sum(2 + 4 * p for p in group_phases)
    n_out = 2 * len(group_phases)
    nbuf = 2
    sec = 14
    nsec = _NCH // sec

    @functools.partial(
        pl.kernel,
        out_type=[jax.ShapeDtypeStruct((_NP, _HD), jnp.float32)] * n_out,
        mesh=_get_mesh(),
        compiler_params=pltpu.CompilerParams(use_tc_tiling_on_sc=False),
        scratch_types=[
            pltpu.VMEM((sec, _CH), jnp.int32),
            pltpu.VMEM((sec, _CH), jnp.int32),
        ]
        + [pltpu.VMEM((_CH, _HD), jnp.float32) for _ in range(nbuf)]
        + [pltpu.SemaphoreType.DMA for _ in range(nbuf)]
        + [pltpu.VMEM_SHARED((_NP, _HD), jnp.float32)],
    )
    def segsum(*refs):
        ins = refs[:n_in]
        outs = refs[n_in:n_in + n_out]
        src_v, dst_v = refs[n_in + n_out], refs[n_in + n_out + 1]
        bufs = refs[n_in + n_out + 2:n_in + n_out + 2 + nbuf]
        sems = refs[n_in + n_out + 2 + nbuf:n_in + n_out + 2 + 2 * nbuf]
        acc = refs[n_in + n_out + 2 + 2 * nbuf]
        c = lax.axis_index("c")
        t = lax.axis_index("s")
        r0 = t * _RPT

        def run(init, phase_refs, out):
            pltpu.sync_copy(init.at[pl.ds(r0, _RPT)], acc.at[pl.ds(r0, _RPT)])
            plsc.subcore_barrier()
            for tab, src, dst in phase_refs:

                def section(s, _):
                    pltpu.sync_copy(src.at[t, pl.ds(s * sec, sec)], src_v)
                    pltpu.sync_copy(dst.at[t, pl.ds(s * sec, sec)], dst_v)

                    def pair(k, _):
                        cp0 = pltpu.async_copy(
                            tab.at[src_v.at[2 * k]], bufs[0], sems[0])
                        cp1 = pltpu.async_copy(
                            tab.at[src_v.at[2 * k + 1]], bufs[1], sems[1])
                        cp0.wait()
                        pltpu.sync_copy(bufs[0], acc.at[dst_v.at[2 * k]],
                                        add=True)
                        cp1.wait()
                        pltpu.sync_copy(bufs[1], acc.at[dst_v.at[2 * k + 1]],
                                        add=True)
                        return 0

                    lax.fori_loop(0, sec // 2, pair, 0)
                    return 0

                lax.fori_loop(0, nsec, section, 0)
            plsc.subcore_barrier()
            pltpu.sync_copy(acc.at[pl.ds(r0, _RPT)], out.at[pl.ds(r0, _RPT)])

        def side(half):
            base = 0
            for g, np_ in enumerate(group_phases):
                init = ins[base + half]
                phase_refs = [
                    (ins[base + 2 + 4 * p + half],
                     ins[base + 2 + 4 * p + 2],
                     ins[base + 2 + 4 * p + 3])
                    for p in range(np_)]
                run(init, phase_refs, outs[2 * g + half])
                base += 2 + 4 * np_

        @pl.when(c == 0)
        def _():
            side(0)

        @pl.when(c == 1)
        def _():
            side(1)

    return segsum


_MPW = (_NB * _BAG) // 32
_MPCH = _MPW // _CH


@functools.lru_cache(maxsize=None)
def _make_mp_gather():
    @functools.partial(
        pl.kernel,
        out_type=[jax.ShapeDtypeStruct((_NB * _BAG, _D), jnp.float32),
                  jax.ShapeDtypeStruct((_NB * _BAG, _D), jnp.float32)],
        mesh=_get_mesh(),
        scratch_types=[
            pltpu.VMEM((_CH,), jnp.int32),
            pltpu.VMEM((_CH, _D), jnp.float32),
            pltpu.SemaphoreType.DMA,
        ],
    )
    def mp_gather(hdf, hsf, idxd, idxs, gd, gs, idx_v, rows_v, sem):
        c = lax.axis_index("c")
        s = lax.axis_index("s")
        wid = s * 2 + c

        def one(tab, idx, out):
            def chunk(i, _):
                off = wid * _MPW + i * _CH
                pltpu.sync_copy(idx.at[pl.ds(off, _CH)], idx_v)
                pltpu.async_copy(tab.at[idx_v], rows_v, sem).wait()
                pltpu.sync_copy(rows_v, out.at[pl.ds(off, _CH)])
                return 0

            lax.fori_loop(0, _MPCH, chunk, 0)

        one(hdf, idxd, gd)
        one(hsf, idxs, gs)

    return mp_gather


def _dot(a, b):
    return jnp.dot(a, b, preferred_element_type=jnp.float32)


def _row_spec(r, ncols=_D):
    return pl.BlockSpec((r, ncols), lambda i: (i, 0))


def _full_spec(shape):
    return pl.BlockSpec(shape, lambda i: tuple(0 for _ in shape))


def _mm_body_first(xd, xs, wld, bld, wls, bls, wdd, wrd, wrr, wsd, wss,
                   h0d, h0s, *outs):
    d = jnp.maximum(_dot(xd[...], wld[...]) + bld[...], 0.0)
    s = jnp.maximum(_dot(xs[...], wls[...]) + bls[...], 0.0)
    h0d[...] = d
    h0s[...] = s
    _mm_common(d, s, wdd, wrd, wrr, wsd, wss, outs)


def _layer_mm_first(xd, xs, wld, bld, wls, bls, wdd, wrd, wrr, wsd, wss):
    return pl.pallas_call(
        _mm_body_first,
        grid=(16,),
        in_specs=[_row_spec(_RPT)] * 2
        + [_full_spec((_D, _D)), _full_spec((1, _D)),
           _full_spec((_D, _D)), _full_spec((1, _D))]
        + [_full_spec((_D, _D))] * 5,
        out_specs=[_row_spec(_RPT)] * 2 + [_row_spec(_RPT, _HD)] * 10,
        out_shape=[jax.ShapeDtypeStruct((_NP, _D), jnp.float32)] * 2
        + [jax.ShapeDtypeStruct((_NP, _HD), jnp.float32)] * 10,
    )(xd, xs, wld, bld, wls, bls, wdd, wrd, wrr, wsd, wss)


def _halves_in(hd, hs):
    def load(x):
        if isinstance(x, tuple):
            return jnp.concatenate(
                [jnp.maximum(x[0][...], 0.0), jnp.maximum(x[1][...], 0.0)],
                axis=1)
        return x[...]
    return load(hd), load(hs)


def _mm_body_full(hd, hs, wdd, wrd, wrr, wsd, wss, *outs):
    _mm_common(hd[...], hs[...], wdd, wrd, wrr, wsd, wss, outs)


def _mm_body_halves(hdl, hdr, hsl, hsr, wdd, wrd, wrr, wsd, wss, *outs):
    d, s = _halves_in((hdl, hdr), (hsl, hsr))
    _mm_common(d, s, wdd, wrd, wrr, wsd, wss, outs)


def _mm_common(d, s, wdd, wrd, wrr, wsd, wss, outs):
    vals = [_dot(s, wdd[...]), _dot(d, wrd[...]), _dot(d, wrr[...]),
            _dot(d, wsd[...]), _dot(s, wss[...])]
    for j, v in enumerate(vals):
        outs[2 * j][...] = v[:, :_HD]
        outs[2 * j + 1][...] = v[:, _HD:]


def _layer_mm(hd, hs, wdd, wrd, wrr, wsd, wss):
    halved = isinstance(hd, tuple)
    body = _mm_body_halves if halved else _mm_body_full
    h_in = ([_row_spec(_RPT, _HD)] * 4 if halved
            else [_row_spec(_RPT), _row_spec(_RPT)])
    h_args = (hd + hs) if halved else (hd, hs)
    return pl.pallas_call(
        body,
        grid=(16,),
        in_specs=h_in + [_full_spec((_D, _D))] * 5,
        out_specs=[_row_spec(_RPT, _HD)] * 10,
        out_shape=[jax.ShapeDtypeStruct((_NP, _HD), jnp.float32)] * 10,
    )(*h_args, wdd, wrd, wrr, wsd, wss)


def _attn_pool(hs, wa, va):
    va_col = va[...].reshape(_D, 1)
    es = [_dot(jnp.tanh(_dot(h, wa[...])), va_col) for h in hs]
    m = jnp.maximum(jnp.maximum(es[0], es[1]), es[2])
    ws = [jnp.exp(e - m) for e in es]
    tot = ws[0] + ws[1] + ws[2]
    return (ws[0] * hs[0] + ws[1] * hs[1] + ws[2] * hs[2]) / tot


def _attn_body(d0, d1l, d1r, d2l, d2r, s0, s1l, s1r, s2l, s2r,
               wad, vad, was, vas, od, os_):
    d1, s1 = _halves_in((d1l, d1r), (s1l, s1r))
    d2, s2 = _halves_in((d2l, d2r), (s2l, s2r))
    od[...] = _attn_pool([d0[...], d1, d2], wad, vad)
    os_[...] = _attn_pool([s0[...], s1, s2], was, vas)


def _layer_attn(d0, d1, d2, s0, s1, s2, wad, vad, was, vas):
    return pl.pallas_call(
        _attn_body,
        grid=(16,),
        in_specs=[_row_spec(_RPT)] + [_row_spec(_RPT, _HD)] * 4
        + [_row_spec(_RPT)] + [_row_spec(_RPT, _HD)] * 4
        + [_full_spec((_D, _D)), _full_spec((1, _D)),
           _full_spec((_D, _D)), _full_spec((1, _D))],
        out_specs=[_row_spec(_RPT)] * 2,
        out_shape=[jax.ShapeDtypeStruct((_NP, _D), jnp.float32)] * 2,
    )(d0, *d1, *d2, s0, *s1, *s2, wad, vad, was, vas)


_MB = 256


def _mil_body(gd, gs, wagg, vmil, wmil, attn_o, bag_o):
    g = (gd[...] + gs[...]).reshape(_MB * _BAG, _D)
    ins = jnp.maximum(_dot(g, wagg[...]), 0.0)
    t3 = jnp.tanh(_dot(ins, vmil[...])).reshape(_MB, _BAG, _D)
    ins3 = ins.reshape(_MB, _BAG, _D)
    w_col = wmil[...].reshape(_D, 1)
    cols = [_dot(t3[:, k, :], w_col) for k in range(_BAG)]
    al = jnp.concatenate(cols, axis=1)
    m = jnp.max(al, axis=1, keepdims=True)
    e = jnp.exp(al - m)
    attn = e / jnp.sum(e, axis=1, keepdims=True)
    attn_o[...] = attn
    bag = attn[:, 0:1] * ins3[:, 0, :]
    for k in range(1, _BAG):
        bag = bag + attn[:, k:k + 1] * ins3[:, k, :]
    bag_o[...] = bag


def _mil(gd3, gs3, wagg, vmil, wmil):
    return pl.pallas_call(
        _mil_body,
        grid=(_NB // _MB,),
        in_specs=[pl.BlockSpec((_MB, _BAG, _D), lambda i: (i, 0, 0))] * 2
        + [_full_spec((_D, _D)), _full_spec((_D, _D)), _full_spec((1, _D))],
        out_specs=[pl.BlockSpec((_MB, _BAG), lambda i: (i, 0)),
                   pl.BlockSpec((_MB, _D), lambda i: (i, 0))],
        out_shape=[jax.ShapeDtypeStruct((_NB, _BAG), jnp.float32),
                   jax.ShapeDtypeStruct((_NB, _D), jnp.float32)],
    )(gd3, gs3, wagg, vmil, wmil)


def _llm_body(x, w, b, bag, w1, b1, w2, b2, o, pred):
    k = pl.program_id(0)

    @pl.when(k == 0)
    def _():
        o[...] = jnp.broadcast_to(b[...], (_NB, _D))

    o[...] += _dot(x[...], w[...])

    @pl.when(k == _NKB - 1)
    def _():
        y = o[...]
        n = jnp.sqrt(jnp.sum(y * y, axis=1, keepdims=True)) + 1e-12
        llm_n = y / n
        kg = bag[...]
        nk = jnp.sqrt(jnp.sum(kg * kg, axis=1, keepdims=True)) + 1e-12
        kgn = kg / nk
        w1v = w1[...]
        h = _dot(kgn, w1v[:_D, :]) + _dot(llm_n, w1v[_D:, :]) + b1[...]
        h = jnp.maximum(h, 0.0)
        p = _dot(h, w2[...]) + b2[0, 0]
        pred[...] = jnp.broadcast_to(p, (_NB, _D))


def _llm_head(x, w, b, bag, w1, b1, w2, b2):
    return pl.pallas_call(
        _llm_body,
        grid=(_NKB,),
        in_specs=[pl.BlockSpec((_NB, _KB), lambda k: (0, k)),
                  pl.BlockSpec((_KB, _D), lambda k: (k, 0)),
                  _full_spec((1, _D)),
                  pl.BlockSpec((_NB, _D), lambda k: (0, 0)),
                  pl.BlockSpec((2 * _D, _D), lambda k: (0, 0)),
                  _full_spec((1, _D)),
                  pl.BlockSpec((_D, 1), lambda k: (0, 0)),
                  _full_spec((1, 1))],
        out_specs=[pl.BlockSpec((_NB, _D), lambda k: (0, 0)),
                   pl.BlockSpec((_NB, _D), lambda k: (0, 0))],
        out_shape=[jax.ShapeDtypeStruct((_NB, _D), jnp.float32),
                   jax.ShapeDtypeStruct((_NB, _D), jnp.float32)],
    )(x, w, b, bag, w1, b1, w2, b2)


def kernel(drug_feat, disease_feat, edge_dd, edge_rd, edge_rr, mp_ins,
           llm_rep, W_lin_drug, b_lin_drug, W_lin_dis, b_lin_dis, W_dd,
           W_rd, W_rr, W_self_drug, W_self_dis, Wa_drug, va_drug, Wa_dis,
           va_dis, W_agg, V_mil, w_mil, W_llm, b_llm, W_mlp1, b_mlp1,
           W_mlp2, b_mlp2):
    rowpad = ((0, _NP - _N), (0, 0))
    dfp = jnp.pad(drug_feat, rowpad)
    sfp = jnp.pad(disease_feat, rowpad)

    def pad_edges(e):
        src = jnp.pad(e[0], (0, _EP - _E)).reshape(16, _NCH, _CH)
        dst = jnp.pad(e[1], (0, _EP - _E),
                      constant_values=_N + 80).reshape(16, _NCH, _CH)
        return src, dst

    dd_s, dd_d = pad_edges(edge_dd)
    rd_s, rd_d = pad_edges(edge_rd)
    rr_s, rr_d = pad_edges(edge_rr)

    row = lambda v: v.reshape(1, -1)

    drugs = []
    diss = []
    hd = hs = None
    for l in range(2):
        if l == 0:
            (h0d, h0s, tddL, tddR, trdL, trdR, trrL, trrR,
             sdL, sdR, ssL, ssR) = _layer_mm_first(
                dfp, sfp, W_lin_drug, row(b_lin_drug),
                W_lin_dis, row(b_lin_dis),
                W_dd[0], W_rd[0], W_rr[0], W_self_drug[0], W_self_dis[0])
            drugs.append(h0d)
            diss.append(h0s)
        else:
            (tddL, tddR, trdL, trdR, trrL, trrR,
             sdL, sdR, ssL, ssR) = _layer_mm(
                hd, hs, W_dd[l], W_rd[l], W_rr[l],
                W_self_drug[l], W_self_dis[l])
        msL, msR = _make_segsum((2,))(
            ssL, ssR, tddL, tddR, dd_s, dd_d, trdL, trdR, rd_s, rd_d)
        mdL, mdR = _make_segsum((1,))(
            sdL, sdR, trrL, trrR, rr_s, rr_d)
        hd, hs = (mdL, mdR), (msL, msR)
        drugs.append(hd)
        diss.append(hs)

    hdf, hsf = _layer_attn(drugs[0], drugs[1], drugs[2],
                           diss[0], diss[1], diss[2],
                           Wa_drug, row(va_drug), Wa_dis, row(va_dis))

    idxd = mp_ins[..., 0].reshape(-1)
    idxs = mp_ins[..., 1].reshape(-1)
    gd, gs = _make_mp_gather()(hdf, hsf, idxd, idxs)

    attn, bag = _mil(gd.reshape(_NB, _BAG, _D), gs.reshape(_NB, _BAG, _D),
                     W_agg, V_mil, row(w_mil))

    _, pred_full = _llm_head(llm_rep, W_llm, row(b_llm), bag,
                             W_mlp1, row(b_mlp1), W_mlp2,
                             b_mlp2.reshape(1, 1))
    return pred_full[:, :1], attn
